# Initial kernel scaffold; baseline (speedup 1.0000x reference)
import jax
import jax.numpy as jnp
from jax.experimental import pallas as pl


def kernel(classifications, regressions, anchors, annotations, imgs, names):
    raise NotImplementedError("write your pallas kernel here")


# speedup vs baseline: 6.9039x; 6.9039x over previous
import functools

import jax
import jax.numpy as jnp
from jax import lax
from jax.experimental import pallas as pl
from jax.experimental.pallas import tpu as pltpu
from jax.experimental.pallas import tpu_sc as plsc

B, A, C, M = 4, 50000, 16, 64
NW = 32
CHUNK = 1568
NSTRIP = CHUNK // 16
LAST_START = A - CHUNK

_LN2 = 0.6931471805599453
_SQRT2 = 1.4142135623730951


def _vlog(x):
    bits = lax.bitcast_convert_type(x, jnp.int32)
    e = jnp.right_shift(bits, 23)
    mbits = (bits & 0x007FFFFF) | 0x3F800000
    m = lax.bitcast_convert_type(mbits, jnp.float32)
    big = m > jnp.float32(_SQRT2)
    m = jnp.where(big, m * 0.5, m)
    ef = (e - 127).astype(jnp.float32) + jnp.where(big, 1.0, 0.0)
    z = (m - 1.0) / (m + 1.0)
    z2 = z * z
    poly = 2.0 + z2 * (0.6666666666666735 + z2 * (0.3999999999940942
           + z2 * (0.2857142874366239 + z2 * 0.22222198432149784)))
    return ef * jnp.float32(_LN2) + z * poly


@functools.partial(
    pl.kernel,
    out_type=jax.ShapeDtypeStruct((NW * 16,), jnp.float32),
    mesh=plsc.VectorSubcoreMesh(core_axis_name="c", subcore_axis_name="s"),
    scratch_types=[
        pltpu.VMEM((CHUNK * 3,), jnp.float32),
        pltpu.VMEM((CHUNK * C,), jnp.float32),
        pltpu.VMEM((CHUNK * 3,), jnp.float32),
        pltpu.VMEM((4 * M,), jnp.float32),
        pltpu.VMEM((16,), jnp.float32),
    ],
    compiler_params=pltpu.CompilerParams(needs_layout_passes=False),
)
def _focal_sc(cls_hbm, reg_hbm, anc_hbm, ann_hbm, out_hbm,
              anc_v, cls_v, reg_v, ann_v, res_v):
    wid = lax.axis_index("s") * 2 + lax.axis_index("c")
    start = jnp.minimum(wid * CHUNK, LAST_START)
    own_lo = wid * CHUNK

    iota = lax.iota(jnp.int32, 16)
    zeros_i = iota * 0

    pltpu.sync_copy(anc_hbm.at[pl.ds(start * 3, CHUNK * 3)], anc_v)

    resvec = jnp.zeros((16,), jnp.float32)
    for j in range(B):
        pltpu.sync_copy(cls_hbm.at[pl.ds(j * A * C + start * C, CHUNK * C)], cls_v)
        pltpu.sync_copy(reg_hbm.at[pl.ds(j * A * 3 + start * 3, CHUNK * 3)], reg_v)
        pltpu.sync_copy(ann_hbm.at[pl.ds(j * 4 * M, 4 * M)], ann_v)

        def strip_body(s, acc):
            cls_acc, npos_acc, xy_acc, ang_acc = acc
            base = s * 16
            aidx = iota * 3 + base * 3
            ax = plsc.load_gather(anc_v, [aidx])
            ay = plsc.load_gather(anc_v, [aidx + 1])
            aal = plsc.load_gather(anc_v, [aidx + 2])

            def m_body(m, mc):
                d2min, bidx = mc
                mv = zeros_i + m
                dx = ax - plsc.load_gather(ann_v, [mv])
                dy = ay - plsc.load_gather(ann_v, [mv + M])
                dd = dx * dx + dy * dy
                lt = dd < d2min
                return jnp.where(lt, dd, d2min), jnp.where(lt, mv, bidx)

            d2min, bidx = lax.fori_loop(
                0, M, m_body,
                (jnp.full((16,), jnp.inf, jnp.float32), zeros_i))

            bx = plsc.load_gather(ann_v, [bidx])
            by = plsc.load_gather(ann_v, [bidx + M])
            bal = plsc.load_gather(ann_v, [bidx + 2 * M])
            blb = plsc.load_gather(ann_v, [bidx + 3 * M])
            aa = jnp.abs(aal - bal)

            validm = (start + base + iota) >= own_lo
            pos = (d2min <= 25.0) & (aa <= 10.0) & validm
            t0 = (d2min >= 56.25) | (aa >= 15.0)
            contrib = (pos | t0) & validm

            npos_acc = npos_acc + jnp.where(pos, 1.0, 0.0)

            cidx = iota * C + base * C
            for c in range(C):
                praw = plsc.load_gather(cls_v, [cidx + c])
                p = jnp.minimum(jnp.maximum(praw, 0.0001), 1.0 - 0.0001)
                is1 = pos & (blb == jnp.float32(c))
                u = jnp.where(is1, 1.0 - p, p)
                v = jnp.where(is1, p, 1.0 - p)
                af = jnp.where(is1, 0.95, 0.05)
                term = af * u * u * _vlog(v)
                cls_acc = cls_acc - jnp.where(contrib, term, 0.0)

            r0 = plsc.load_gather(reg_v, [aidx])
            r1 = plsc.load_gather(reg_v, [aidx + 1])
            r2 = plsc.load_gather(reg_v, [aidx + 2])
            dxr = jnp.abs((bx - ax) - r0)
            dyr = jnp.abs((by - ay) - r1)
            lx = jnp.where(dxr <= 1.0 / 9.0, 4.5 * dxr * dxr, dxr - 0.5 / 9.0)
            ly = jnp.where(dyr <= 1.0 / 9.0, 4.5 * dyr * dyr, dyr - 0.5 / 9.0)
            da = (jnp.abs((bal - aal) - r2) - 10.0) / 5.0
            da = jnp.where(da <= 0.0, 0.0, da)
            posf = jnp.where(pos, 1.0, 0.0)
            xy_acc = xy_acc + (lx + ly) * posf
            ang_acc = ang_acc + da * posf
            return cls_acc, npos_acc, xy_acc, ang_acc

        zf = jnp.zeros((16,), jnp.float32)
        cls_acc, npos_acc, xy_acc, ang_acc = lax.fori_loop(
            0, NSTRIP, strip_body, (zf, zf, zf, zf))

        resvec = jnp.where(iota == 4 * j, jnp.sum(cls_acc), resvec)
        resvec = jnp.where(iota == 4 * j + 1, jnp.sum(npos_acc), resvec)
        resvec = jnp.where(iota == 4 * j + 2, jnp.sum(xy_acc), resvec)
        resvec = jnp.where(iota == 4 * j + 3, jnp.sum(ang_acc), resvec)

    res_v[...] = resvec
    pltpu.sync_copy(res_v, out_hbm.at[pl.ds(wid * 16, 16)])


def kernel(classifications, regressions, anchors, annotations, imgs, names):
    cls_flat = classifications.reshape(-1)
    reg_flat = regressions.reshape(-1)
    anc_flat = anchors.reshape(-1)
    ann_t = jnp.transpose(annotations, (0, 2, 1)).reshape(-1)
    out = _focal_sc(cls_flat, reg_flat, anc_flat, ann_t)
    parts = out.reshape(NW, B, 4).sum(axis=0)
    npos = parts[:, 1]
    cls = parts[:, 0] / jnp.maximum(npos, 1.0)
    xy = parts[:, 2] / jnp.maximum(2.0 * npos, 1.0)
    ang = parts[:, 3] / jnp.maximum(npos, 1.0)
    return (cls.mean(keepdims=True), xy.mean(keepdims=True),
            ang.mean(keepdims=True))


# speedup vs baseline: 13.1955x; 1.9113x over previous
import functools

import jax
import jax.numpy as jnp
from jax import lax
from jax.experimental import pallas as pl
from jax.experimental.pallas import tpu as pltpu
from jax.experimental.pallas import tpu_sc as plsc

B, A, C, M = 4, 50000, 16, 64
NW = 32
CHUNK = 1568
NSTRIP = CHUNK // 16
LAST_START = A - CHUNK

_LN2 = 0.6931471805599453
_C5 = [0.029808765243528598, -0.2790010238760822, 1.1017396261345287,
       -2.418999477903287, 3.4989067477007527, -1.9324431902018802]


def _vlog(x):
    bits = lax.bitcast_convert_type(x, jnp.int32)
    ef = (jnp.right_shift(bits, 23) - 127).astype(jnp.float32)
    m = lax.bitcast_convert_type((bits & 0x007FFFFF) | 0x3F800000, jnp.float32)
    p = jnp.float32(_C5[0])
    for c in _C5[1:]:
        p = p * m + jnp.float32(c)
    return ef * jnp.float32(_LN2) + p


@functools.partial(
    pl.kernel,
    out_type=jax.ShapeDtypeStruct((NW * 16,), jnp.float32),
    mesh=plsc.VectorSubcoreMesh(core_axis_name="c", subcore_axis_name="s"),
    scratch_types=[
        pltpu.VMEM((CHUNK,), jnp.float32),
        pltpu.VMEM((CHUNK,), jnp.float32),
        pltpu.VMEM((CHUNK,), jnp.float32),
        pltpu.VMEM((CHUNK * C,), jnp.float32),
        pltpu.VMEM((CHUNK,), jnp.float32),
        pltpu.VMEM((CHUNK,), jnp.float32),
        pltpu.VMEM((CHUNK,), jnp.float32),
        pltpu.VMEM((4 * M,), jnp.float32),
        pltpu.VMEM((16,), jnp.float32),
    ],
    compiler_params=pltpu.CompilerParams(needs_layout_passes=False),
)
def _focal_sc(cls_hbm, regt_hbm, anct_hbm, ann_hbm, out_hbm,
              ax_v, ay_v, aal_v, cls_v, r0_v, r1_v, r2_v, ann_v, res_v):
    wid = lax.axis_index("s") * 2 + lax.axis_index("c")
    start = jnp.minimum(wid * CHUNK, LAST_START)
    own_lo = wid * CHUNK

    iota = lax.iota(jnp.int32, 16)
    zeros_i = iota * 0

    pltpu.sync_copy(anct_hbm.at[pl.ds(start, CHUNK)], ax_v)
    pltpu.sync_copy(anct_hbm.at[pl.ds(A + start, CHUNK)], ay_v)
    pltpu.sync_copy(anct_hbm.at[pl.ds(2 * A + start, CHUNK)], aal_v)

    def strip_tail(base, d2min, bidx, acc):
        cls_acc, npos_acc, xy_acc, ang_acc = acc
        aal = aal_v[pl.ds(base, 16)]
        bx = plsc.load_gather(ann_v, [bidx])
        by = plsc.load_gather(ann_v, [bidx + M])
        bal = plsc.load_gather(ann_v, [bidx + 2 * M])
        blb = plsc.load_gather(ann_v, [bidx + 3 * M])
        aa = jnp.abs(aal - bal)

        validm = (start + base + iota) >= own_lo
        pos = (d2min <= 25.0) & (aa <= 10.0) & validm
        t0 = (d2min >= 56.25) | (aa >= 15.0)
        contrib = (pos | t0) & validm
        npos_acc = npos_acc + jnp.where(pos, 1.0, 0.0)

        cidx = iota * C + base * C
        s_acc = jnp.zeros((16,), jnp.float32)
        for c in range(C):
            praw = plsc.load_gather(cls_v, [cidx + c])
            p = jnp.minimum(jnp.maximum(praw, 0.0001), 1.0 - 0.0001)
            s_acc = s_acc - (p * p) * _vlog(1.0 - p)
        cls_acc = cls_acc + jnp.where(contrib, 0.05 * s_acc, 0.0)

        labi = blb.astype(jnp.int32)
        plab = plsc.load_gather(cls_v, [cidx + labi])
        plab = jnp.minimum(jnp.maximum(plab, 0.0001), 1.0 - 0.0001)
        vlab = 1.0 - plab
        corr = (-0.95) * vlab * vlab * _vlog(plab) + \
               (0.05 * plab * plab) * _vlog(vlab)
        cls_acc = cls_acc + jnp.where(pos, corr, 0.0)

        ax = ax_v[pl.ds(base, 16)]
        ay = ay_v[pl.ds(base, 16)]
        r0 = r0_v[pl.ds(base, 16)]
        r1 = r1_v[pl.ds(base, 16)]
        r2 = r2_v[pl.ds(base, 16)]
        dxr = jnp.abs((bx - ax) - r0)
        dyr = jnp.abs((by - ay) - r1)
        lx = jnp.where(dxr <= 1.0 / 9.0, 4.5 * dxr * dxr, dxr - 0.5 / 9.0)
        ly = jnp.where(dyr <= 1.0 / 9.0, 4.5 * dyr * dyr, dyr - 0.5 / 9.0)
        da = (jnp.abs((bal - aal) - r2) - 10.0) / 5.0
        da = jnp.where(da <= 0.0, 0.0, da)
        posf = jnp.where(pos, 1.0, 0.0)
        xy_acc = xy_acc + (lx + ly) * posf
        ang_acc = ang_acc + da * posf
        return cls_acc, npos_acc, xy_acc, ang_acc

    def batch_body(j, resvec):
        pltpu.sync_copy(cls_hbm.at[pl.ds(j * (A * C) + start * C, CHUNK * C)], cls_v)
        pltpu.sync_copy(regt_hbm.at[pl.ds(j * (3 * A) + start, CHUNK)], r0_v)
        pltpu.sync_copy(regt_hbm.at[pl.ds(j * (3 * A) + A + start, CHUNK)], r1_v)
        pltpu.sync_copy(regt_hbm.at[pl.ds(j * (3 * A) + 2 * A + start, CHUNK)], r2_v)
        pltpu.sync_copy(ann_hbm.at[pl.ds(j * (4 * M), 4 * M)], ann_v)

        def group_body(g, acc):
            base0 = g * 32
            base1 = base0 + 16
            ax0 = ax_v[pl.ds(base0, 16)]
            ay0 = ay_v[pl.ds(base0, 16)]
            ax1 = ax_v[pl.ds(base1, 16)]
            ay1 = ay_v[pl.ds(base1, 16)]

            def m_body(m, mc):
                d0, b0, d1, b1 = mc
                mv = zeros_i + m
                gx = plsc.load_gather(ann_v, [mv])
                gy = plsc.load_gather(ann_v, [mv + M])
                dx0 = ax0 - gx
                dy0 = ay0 - gy
                dd0 = dx0 * dx0 + dy0 * dy0
                dx1 = ax1 - gx
                dy1 = ay1 - gy
                dd1 = dx1 * dx1 + dy1 * dy1
                lt0 = dd0 < d0
                lt1 = dd1 < d1
                return (jnp.where(lt0, dd0, d0), jnp.where(lt0, mv, b0),
                        jnp.where(lt1, dd1, d1), jnp.where(lt1, mv, b1))

            inf = jnp.full((16,), jnp.inf, jnp.float32)
            d0, b0, d1, b1 = lax.fori_loop(0, M, m_body,
                                           (inf, zeros_i, inf, zeros_i))
            acc = strip_tail(base0, d0, b0, acc)
            acc = strip_tail(base1, d1, b1, acc)
            return acc

        zf = jnp.zeros((16,), jnp.float32)
        cls_acc, npos_acc, xy_acc, ang_acc = lax.fori_loop(
            0, NSTRIP // 2, group_body, (zf, zf, zf, zf))

        resvec = jnp.where(iota == 4 * j, jnp.sum(cls_acc), resvec)
        resvec = jnp.where(iota == 4 * j + 1, jnp.sum(npos_acc), resvec)
        resvec = jnp.where(iota == 4 * j + 2, jnp.sum(xy_acc), resvec)
        resvec = jnp.where(iota == 4 * j + 3, jnp.sum(ang_acc), resvec)
        return resvec

    res_v[...] = lax.fori_loop(0, B, batch_body, jnp.zeros((16,), jnp.float32))
    pltpu.sync_copy(res_v, out_hbm.at[pl.ds(wid * 16, 16)])


def kernel(classifications, regressions, anchors, annotations, imgs, names):
    cls_flat = classifications.reshape(-1)
    regt = jnp.transpose(regressions, (0, 2, 1)).reshape(-1)
    anct = jnp.transpose(anchors[0], (1, 0)).reshape(-1)
    ann_t = jnp.transpose(annotations, (0, 2, 1)).reshape(-1)
    out = _focal_sc(cls_flat, regt, anct, ann_t)
    parts = out.reshape(NW, B, 4).sum(axis=0)
    npos = parts[:, 1]
    cls = parts[:, 0] / jnp.maximum(npos, 1.0)
    xy = parts[:, 2] / jnp.maximum(2.0 * npos, 1.0)
    ang = parts[:, 3] / jnp.maximum(npos, 1.0)
    return (cls.mean(keepdims=True), xy.mean(keepdims=True),
            ang.mean(keepdims=True))
